# baseline (speedup 1.0000x reference)
import functools

import jax
import jax.numpy as jnp
from jax import lax
from jax.experimental import pallas as pl
from jax.experimental.pallas import tpu as pltpu
from jax.experimental.pallas import tpu_sc as plsc

VOCAB = 100000
DIM = 128
LABELS = 1000
B = 4096
L = 200

NC = 2
NS = 16
NW = NC * NS
DOCS_PER_W = B // NW
LANES = 16
NSEG = DIM // LANES


def _sc_gather_sum(x, table):
    mesh = plsc.VectorSubcoreMesh(core_axis_name="c", subcore_axis_name="s")

    NBUF = 3

    @functools.partial(
        pl.kernel,
        mesh=mesh,
        out_type=jax.ShapeDtypeStruct((B, DIM), jnp.float32),
        scratch_types=[
            pltpu.VMEM((DOCS_PER_W, L), jnp.int32),
            [pltpu.VMEM((L, DIM), jnp.float32)] * NBUF,
            pltpu.VMEM((DOCS_PER_W, DIM), jnp.float32),
            [pltpu.SemaphoreType.DMA] * NBUF,
        ],
    )
    def k(x_hbm, table_hbm, out_hbm, idx_v, rows, out_v, gsem):
        wid = lax.axis_index("s") * NC + lax.axis_index("c")
        base = wid * DOCS_PER_W
        L2 = L - 128

        pltpu.sync_copy(x_hbm.at[pl.ds(base, DOCS_PER_W)], idx_v)

        CHUNKS = ((0, 128), (128, L2))

        def fire(d, i):
            for (o, n) in CHUNKS:
                pltpu.async_copy(table_hbm.at[idx_v.at[d, pl.ds(o, n)]],
                                 rows[i].at[pl.ds(o, n)], gsem[i])

        def reduce_rows(buf, start, n, carry0):
            @plsc.parallel_loop(start, start + n, 1, unroll=8, carry=carry0)
            def acc(r, carry):
                return tuple(
                    carry[j] + buf[r, pl.ds(j * LANES, LANES)]
                    for j in range(NSEG))
            return acc

        def consume(d, i):
            zero = (jnp.zeros((LANES,), jnp.float32),) * NSEG
            for (o, n) in CHUNKS:
                pltpu.make_async_copy(table_hbm.at[pl.ds(0, n)],
                                      rows[i].at[pl.ds(o, n)],
                                      gsem[i]).wait()
            acc = reduce_rows(rows[i], 0, L, zero)
            for j in range(NSEG):
                out_v[d, pl.ds(j * LANES, LANES)] = acc[j]

        fire(0, 0)
        fire(1, 1)

        @pl.loop(0, 42)
        def _(g):
            a = 3 * g
            fire(a + 2, 2)
            consume(a, 0)
            fire(a + 3, 0)
            consume(a + 1, 1)
            fire(a + 4, 1)
            consume(a + 2, 2)

        consume(126, 0)
        consume(127, 1)

        pltpu.sync_copy(out_v, out_hbm.at[pl.ds(base, DOCS_PER_W)])

    return k(x, table)


def _tc_binarize_matmul(doc_sum, W, b):
    BBLK = 512
    b2 = b.reshape(1, LABELS)

    def body(e_ref, w_ref, b_ref, o_ref):
        e = (e_ref[...] > 0.0).astype(jnp.float32)
        o_ref[...] = lax.dot_general(
            e, w_ref[...], (((1,), (1,)), ((), ())),
            preferred_element_type=jnp.float32,
            precision=lax.Precision.HIGHEST) + b_ref[...]

    return pl.pallas_call(
        body,
        grid=(B // BBLK,),
        in_specs=[
            pl.BlockSpec((BBLK, DIM), lambda i: (i, 0)),
            pl.BlockSpec((LABELS, DIM), lambda i: (0, 0)),
            pl.BlockSpec((1, LABELS), lambda i: (0, 0)),
        ],
        out_specs=pl.BlockSpec((BBLK, LABELS), lambda i: (i, 0)),
        out_shape=jax.ShapeDtypeStruct((B, LABELS), jnp.float32),
    )(doc_sum, W, b2)


def kernel(x, m, table, W, b):
    del m
    doc_sum = _sc_gather_sum(x, table)
    return _tc_binarize_matmul(doc_sum, W, b)


# speedup vs baseline: 1.0419x; 1.0419x over previous
import functools

import jax
import jax.numpy as jnp
from jax import lax
from jax.experimental import pallas as pl
from jax.experimental.pallas import tpu as pltpu
from jax.experimental.pallas import tpu_sc as plsc

VOCAB = 100000
DIM = 128
LABELS = 1000
B = 4096
L = 200

NC = 2
NS = 16
NW = NC * NS
DOCS_PER_W = B // NW
LANES = 16
NSEG = DIM // LANES


def _sc_gather_sum(x, table):
    mesh = plsc.VectorSubcoreMesh(core_axis_name="c", subcore_axis_name="s")

    NBUF = 3

    @functools.partial(
        pl.kernel,
        mesh=mesh,
        out_type=jax.ShapeDtypeStruct((B, DIM), jnp.float32),
        scratch_types=[
            pltpu.VMEM((DOCS_PER_W, L), jnp.int32),
            [pltpu.VMEM((L, DIM), jnp.float32)] * NBUF,
            pltpu.VMEM((DOCS_PER_W, DIM), jnp.float32),
            [pltpu.SemaphoreType.DMA] * NBUF,
        ],
    )
    def k(x_hbm, table_hbm, out_hbm, idx_v, rows, out_v, gsem):
        wid = lax.axis_index("s") * NC + lax.axis_index("c")
        base = wid * DOCS_PER_W
        L2 = L - 128

        pltpu.sync_copy(x_hbm.at[pl.ds(base, DOCS_PER_W)], idx_v)

        CHUNKS = ((0, 128), (128, L2))

        def fire(d, i):
            for (o, n) in CHUNKS:
                pltpu.async_copy(table_hbm.at[idx_v.at[d, pl.ds(o, n)]],
                                 rows[i].at[pl.ds(o, n)], gsem[i])

        def reduce_rows(buf, start, n, carry0):
            @plsc.parallel_loop(start, start + n, 1, unroll=8, carry=carry0)
            def acc(r, carry):
                return tuple(
                    carry[j] + buf[r, pl.ds(j * LANES, LANES)]
                    for j in range(NSEG))
            return acc

        def consume(d, i):
            zero = (jnp.zeros((LANES,), jnp.float32),) * NSEG
            for (o, n) in CHUNKS:
                pltpu.make_async_copy(table_hbm.at[pl.ds(0, n)],
                                      rows[i].at[pl.ds(o, n)],
                                      gsem[i]).wait()
            acc = reduce_rows(rows[i], 0, L, zero)
            for j in range(NSEG):
                out_v[d, pl.ds(j * LANES, LANES)] = acc[j]

        fire(0, 0)
        fire(1, 1)

        @pl.loop(0, 42)
        def _(g):
            a = 3 * g
            fire(a + 2, 2)
            consume(a, 0)
            fire(a + 3, 0)
            consume(a + 1, 1)
            fire(a + 4, 1)
            consume(a + 2, 2)

        consume(126, 0)
        consume(127, 1)

        pltpu.sync_copy(out_v, out_hbm.at[pl.ds(base, DOCS_PER_W)])

    return k(x, table)


def _tc_binarize_matmul(doc_sum, W, b):
    LB = 1024
    Wp = jnp.zeros((LB, DIM), jnp.float32).at[:LABELS].set(W)
    bp = jnp.zeros((1, LB), jnp.float32).at[0, :LABELS].set(b)
    BBLK = 512

    def body(e_ref, w_ref, b_ref, o_ref):
        e = (e_ref[...] > 0.0).astype(jnp.float32)
        o_ref[...] = lax.dot_general(
            e, w_ref[...], (((1,), (1,)), ((), ())),
            preferred_element_type=jnp.float32) + b_ref[...]

    out = pl.pallas_call(
        body,
        grid=(B // BBLK,),
        in_specs=[
            pl.BlockSpec((BBLK, DIM), lambda i: (i, 0)),
            pl.BlockSpec((LB, DIM), lambda i: (0, 0)),
            pl.BlockSpec((1, LB), lambda i: (0, 0)),
        ],
        out_specs=pl.BlockSpec((BBLK, LB), lambda i: (i, 0)),
        out_shape=jax.ShapeDtypeStruct((B, LB), jnp.float32),
    )(doc_sum, Wp, bp)
    return out[:, :LABELS]


def kernel(x, m, table, W, b):
    del m
    doc_sum = _sc_gather_sum(x, table)
    return _tc_binarize_matmul(doc_sum, W, b)
